# baseline (speedup 1.0000x reference)
import jax
import jax.numpy as jnp
from jax import lax
from jax.experimental import pallas as pl
from jax.experimental.pallas import tpu as pltpu
from jax.experimental.pallas import tpu_sc as plsc
from functools import partial

_B, _S, _H = 32, 8, 1024
_E = 64
_TOP_K = 8
_N_GROUP = 8
_TOPK_GROUP = 4
_I_MOE = 512
_I_SHARED = 2048
_SCALING = 2.5
_T = _B * _S
_GSZ = _E // _N_GROUP
_SH_STEPS = 16
_SH_CHUNK = _I_SHARED // _SH_STEPS
_EPG = 4
_NEG = -1e30
_L = 16
_BATCHES = _T // _L


def _scores_body(x_ref, rw_ref, eb_ref, sc_ref, sfc_ref):
    logits = jax.lax.dot_general(
        rw_ref[...], x_ref[...], (((1,), (1,)), ((), ())),
        preferred_element_type=jnp.float32)
    scores = jax.nn.sigmoid(logits)
    sfc = scores + eb_ref[...]
    for b in range(_BATCHES):
        sc_ref[b] = scores[:, b * _L:(b + 1) * _L]
        sfc_ref[b] = sfc[:, b * _L:(b + 1) * _L]


def _sc_route_body(sc_hbm, sfc_hbm, comb_hbm, sc_v, sfc_v, out_v, sem):
    wid = lax.axis_index("s") * 2 + lax.axis_index("c")

    @pl.when(wid < _BATCHES)
    def _():
        pltpu.sync_copy(sc_hbm.at[wid], sc_v)
        pltpu.sync_copy(sfc_hbm.at[wid], sfc_v)

        scores = [sc_v[e, :] for e in range(_E)]
        sfc = [sfc_v[e, :] for e in range(_E)]

        group_scores = []
        for g in range(_N_GROUP):
            seg = sfc[g * _GSZ:(g + 1) * _GSZ]
            m1 = seg[0]
            for s in seg[1:]:
                m1 = jnp.maximum(m1, s)
            fi = jnp.full((_L,), 127, jnp.int32)
            for j, s in enumerate(seg):
                fi = jnp.minimum(fi, jnp.where(s == m1, j, 127))
            m2 = jnp.full((_L,), _NEG, jnp.float32)
            for j, s in enumerate(seg):
                m2 = jnp.maximum(m2, jnp.where(fi == j, _NEG, s))
            group_scores.append(m1 + m2)

        gmask = [jnp.zeros((_L,), jnp.float32) for _ in range(_N_GROUP)]
        gtmp = list(group_scores)
        for _i in range(_TOPK_GROUP):
            m = gtmp[0]
            for s in gtmp[1:]:
                m = jnp.maximum(m, s)
            fi = jnp.full((_L,), 127, jnp.int32)
            for j, s in enumerate(gtmp):
                fi = jnp.minimum(fi, jnp.where(s == m, j, 127))
            for j in range(_N_GROUP):
                sel = fi == j
                gmask[j] = jnp.where(sel, 1.0, gmask[j])
                gtmp[j] = jnp.where(sel, _NEG, gtmp[j])

        masked = [jnp.where(gmask[e // _GSZ] > 0, sfc[e], 0.0)
                  for e in range(_E)]

        fis = []
        ws = []
        wsum = jnp.zeros((_L,), jnp.float32)
        for _i in range(_TOP_K):
            m = masked[0]
            for s in masked[1:]:
                m = jnp.maximum(m, s)
            fi = jnp.full((_L,), 9999, jnp.int32)
            for j, s in enumerate(masked):
                fi = jnp.minimum(fi, jnp.where(s == m, j, 9999))
            w = jnp.zeros((_L,), jnp.float32)
            for j in range(_E):
                w = w + jnp.where(fi == j, scores[j], 0.0)
            for j in range(_E):
                masked[j] = jnp.where(fi == j, _NEG, masked[j])
            fis.append(fi)
            ws.append(w)
            wsum = wsum + w

        scale = _SCALING / (wsum + 1e-20)
        for e in range(_E):
            acc = jnp.zeros((_L,), jnp.float32)
            for fi, w in zip(fis, ws):
                acc = acc + jnp.where(fi == e, w, 0.0)
            out_v[e, :] = acc * scale
        pltpu.sync_copy(out_v, comb_hbm.at[wid])


def _moe_body(x_ref, comb_ref, up_ref, dn_ref, su_ref, sd_ref, out_ref):
    e = pl.program_id(0)
    x = x_ref[...]

    @pl.when(e == 0)
    def _init():
        out_ref[...] = jnp.zeros_like(out_ref)

    xb = x.astype(jnp.bfloat16)

    @pl.when(e < _SH_STEPS)
    def _shared():
        hs = jnp.maximum(jax.lax.dot_general(
            xb, su_ref[...].astype(jnp.bfloat16), (((1,), (1,)), ((), ())),
            preferred_element_type=jnp.float32), 0.0)
        out_ref[...] += jax.lax.dot_general(
            hs.astype(jnp.bfloat16), sd_ref[...].astype(jnp.bfloat16),
            (((1,), (1,)), ((), ())),
            preferred_element_type=jnp.float32)

    sub = jax.lax.broadcasted_iota(jnp.int32, (_E, 1), 0)
    comb_t = jnp.concatenate([comb_ref[b] for b in range(_BATCHES)], axis=1)
    acc = out_ref[...]
    for j in range(_EPG):
        ej = e * _EPG + j
        onehot = (sub == ej).astype(jnp.float32)
        c = jax.lax.dot_general(
            comb_t, onehot, (((0,), (0,)), ((), ())),
            preferred_element_type=jnp.float32)
        h = jnp.maximum(jax.lax.dot_general(
            xb, up_ref[j].astype(jnp.bfloat16), (((1,), (1,)), ((), ())),
            preferred_element_type=jnp.float32), 0.0)
        acc += jax.lax.dot_general(
            (h * c).astype(jnp.bfloat16), dn_ref[j].astype(jnp.bfloat16),
            (((1,), (1,)), ((), ())),
            preferred_element_type=jnp.float32)
    out_ref[...] = acc


def kernel(hidden_states, router_weight, up_w, down_w,
           shared_up_w, shared_down_w, e_bias):
    x = hidden_states.reshape(_T, _H)
    eb = e_bias.reshape(_E, 1)

    scores_t, sfc_t = pl.pallas_call(
        _scores_body,
        in_specs=[pl.BlockSpec((_T, _H), lambda: (0, 0)),
                  pl.BlockSpec((_E, _H), lambda: (0, 0)),
                  pl.BlockSpec((_E, 1), lambda: (0, 0))],
        out_specs=[pl.BlockSpec((_BATCHES, _E, _L), lambda: (0, 0, 0)),
                   pl.BlockSpec((_BATCHES, _E, _L), lambda: (0, 0, 0))],
        out_shape=[jax.ShapeDtypeStruct((_BATCHES, _E, _L), jnp.float32),
                   jax.ShapeDtypeStruct((_BATCHES, _E, _L), jnp.float32)],
    )(x, router_weight, eb)

    mesh = plsc.VectorSubcoreMesh(core_axis_name="c", subcore_axis_name="s")
    combine_t = pl.kernel(
        _sc_route_body,
        mesh=mesh,
        out_type=jax.ShapeDtypeStruct((_BATCHES, _E, _L), jnp.float32),
        scratch_types=[
            pltpu.VMEM((_E, _L), jnp.float32),
            pltpu.VMEM((_E, _L), jnp.float32),
            pltpu.VMEM((_E, _L), jnp.float32),
            pltpu.SemaphoreType.DMA,
        ],
    )(scores_t, sfc_t)

    out = pl.pallas_call(
        _moe_body,
        grid=(_E // _EPG,),
        in_specs=[
            pl.BlockSpec((_T, _H), lambda e: (0, 0)),
            pl.BlockSpec((_BATCHES, _E, _L), lambda e: (0, 0, 0)),
            pl.BlockSpec((_EPG, _I_MOE, _H), lambda e: (e, 0, 0)),
            pl.BlockSpec((_EPG, _H, _I_MOE), lambda e: (e, 0, 0)),
            pl.BlockSpec((_SH_CHUNK, _H),
                         lambda e: (jnp.minimum(e, _SH_STEPS - 1), 0)),
            pl.BlockSpec((_H, _SH_CHUNK),
                         lambda e: (0, jnp.minimum(e, _SH_STEPS - 1))),
        ],
        out_specs=pl.BlockSpec((_T, _H), lambda e: (0, 0)),
        out_shape=jax.ShapeDtypeStruct((_T, _H), jnp.float32),
    )(x, combine_t, up_w, down_w, shared_up_w, shared_down_w)

    return out.reshape(_B, _S, _H)


# speedup vs baseline: 1.2622x; 1.2622x over previous
import jax
import jax.numpy as jnp
from jax.experimental import pallas as pl
from jax.experimental.pallas import tpu as pltpu
from functools import partial

_B, _S, _H = 32, 8, 1024
_E = 64
_TOP_K = 8
_N_GROUP = 8
_TOPK_GROUP = 4
_I_MOE = 512
_I_SHARED = 2048
_SCALING = 2.5
_T = _B * _S
_GSZ = _E // _N_GROUP
_SH_FIRST = 8
_SH_STEPS = 8
_SH_CHUNK = _I_SHARED // _SH_STEPS

_NEG = -1e30


def _routing(x, rw, eb):
    logits = jax.lax.dot_general(
        x, rw, (((1,), (1,)), ((), ())), preferred_element_type=jnp.float32)
    scores = jax.nn.sigmoid(logits)
    sfc = scores + eb
    lane = jax.lax.broadcasted_iota(jnp.int32, (_T, _E), 1)

    gs = []
    for g in range(_N_GROUP):
        seg = sfc[:, g * _GSZ:(g + 1) * _GSZ]
        il = jax.lax.broadcasted_iota(jnp.int32, (_T, _GSZ), 1)
        m1 = jnp.max(seg, axis=1, keepdims=True)
        fi = jnp.min(jnp.where(seg == m1, il, 127), axis=1, keepdims=True)
        m2 = jnp.max(jnp.where(il == fi, _NEG, seg), axis=1, keepdims=True)
        gs.append(m1 + m2)
    group_scores = jnp.concatenate(gs, axis=1)

    gil = jax.lax.broadcasted_iota(jnp.int32, (_T, _N_GROUP), 1)
    gmask = jnp.zeros((_T, _N_GROUP), jnp.float32)
    gtmp = group_scores
    for _ in range(_TOPK_GROUP):
        m = jnp.max(gtmp, axis=1, keepdims=True)
        fi = jnp.min(jnp.where(gtmp == m, gil, 127), axis=1, keepdims=True)
        sel = gil == fi
        gmask = jnp.where(sel, 1.0, gmask)
        gtmp = jnp.where(sel, _NEG, gtmp)

    smask = jnp.concatenate(
        [jnp.broadcast_to(gmask[:, g:g + 1], (_T, _GSZ)) for g in range(_N_GROUP)],
        axis=1)
    masked = jnp.where(smask > 0, sfc, 0.0)

    comb = jnp.zeros((_T, _E), jnp.float32)
    wsum = jnp.zeros((_T, 1), jnp.float32)
    for _ in range(_TOP_K):
        m = jnp.max(masked, axis=1, keepdims=True)
        fi = jnp.min(jnp.where(masked == m, lane, 9999), axis=1, keepdims=True)
        sel = lane == fi
        w = jnp.sum(jnp.where(sel, scores, 0.0), axis=1, keepdims=True)
        comb = comb + jnp.where(sel, w, 0.0)
        wsum = wsum + w
        masked = jnp.where(sel, _NEG, masked)
    return comb * (_SCALING / (wsum + 1e-20))


_EPG = 4


def _moe_body(x_ref, rw_ref, eb_ref, up_ref, dn_ref,
              su_ref, sd_ref, out_ref, comb_ref):
    e = pl.program_id(0)
    x = x_ref[...]

    @pl.when(e == 0)
    def _init():
        comb_ref[...] = _routing(x, rw_ref[...], eb_ref[...])
        out_ref[...] = jnp.zeros_like(out_ref)

    xb = x.astype(jnp.bfloat16)

    @pl.when(e >= _SH_FIRST)
    def _shared():
        hs = jnp.maximum(jax.lax.dot_general(
            xb, su_ref[...].astype(jnp.bfloat16), (((1,), (1,)), ((), ())),
            preferred_element_type=jnp.float32), 0.0)
        out_ref[...] += jax.lax.dot_general(
            hs.astype(jnp.bfloat16), sd_ref[...].astype(jnp.bfloat16),
            (((1,), (1,)), ((), ())),
            preferred_element_type=jnp.float32)

    lane = jax.lax.broadcasted_iota(jnp.int32, (_T, _E), 1)
    acc = out_ref[...]
    for j in range(_EPG):
        ej = e * _EPG + j
        c = jnp.sum(jnp.where(lane == ej, comb_ref[...], 0.0),
                    axis=1, keepdims=True)
        h = jnp.maximum(jax.lax.dot_general(
            xb, up_ref[j].astype(jnp.bfloat16), (((1,), (1,)), ((), ())),
            preferred_element_type=jnp.float32), 0.0)
        acc += jax.lax.dot_general(
            (h * c).astype(jnp.bfloat16), dn_ref[j].astype(jnp.bfloat16),
            (((1,), (1,)), ((), ())),
            preferred_element_type=jnp.float32)
    out_ref[...] = acc


def kernel(hidden_states, router_weight, up_w, down_w,
           shared_up_w, shared_down_w, e_bias):
    x = hidden_states.reshape(_T, _H)
    eb = e_bias.reshape(1, _E)

    out = pl.pallas_call(
        _moe_body,
        grid=(_E // _EPG,),
        in_specs=[
            pl.BlockSpec((_T, _H), lambda e: (0, 0)),
            pl.BlockSpec((_E, _H), lambda e: (0, 0)),
            pl.BlockSpec((1, _E), lambda e: (0, 0)),
            pl.BlockSpec((_EPG, _I_MOE, _H), lambda e: (e, 0, 0)),
            pl.BlockSpec((_EPG, _H, _I_MOE), lambda e: (e, 0, 0)),
            pl.BlockSpec((_SH_CHUNK, _H),
                         lambda e: (jnp.clip(e - _SH_FIRST, 0,
                                             _SH_STEPS - 1), 0)),
            pl.BlockSpec((_H, _SH_CHUNK),
                         lambda e: (0, jnp.clip(e - _SH_FIRST, 0,
                                                _SH_STEPS - 1))),
        ],
        out_specs=pl.BlockSpec((_T, _H), lambda e: (0, 0)),
        out_shape=jax.ShapeDtypeStruct((_T, _H), jnp.float32),
        scratch_shapes=[pltpu.VMEM((_T, _E), jnp.float32)],
    )(x, router_weight, eb, up_w, down_w, shared_up_w, shared_down_w)

    return out.reshape(_B, _S, _H)


# speedup vs baseline: 1.2677x; 1.0044x over previous
import jax
import jax.numpy as jnp
from jax.experimental import pallas as pl
from jax.experimental.pallas import tpu as pltpu
from functools import partial

_B, _S, _H = 32, 8, 1024
_E = 64
_TOP_K = 8
_N_GROUP = 8
_TOPK_GROUP = 4
_I_MOE = 512
_I_SHARED = 2048
_SCALING = 2.5
_T = _B * _S
_GSZ = _E // _N_GROUP
_SH_FIRST = 1
_SH_STEPS = 8
_SH_CHUNK = _I_SHARED // _SH_STEPS

_NEG = -1e30


def _routing(x, rw, eb):
    logits = jax.lax.dot_general(
        x, rw, (((1,), (1,)), ((), ())), preferred_element_type=jnp.float32)
    scores = jax.nn.sigmoid(logits)
    sfc = scores + eb
    lane = jax.lax.broadcasted_iota(jnp.int32, (_T, _E), 1)

    gs = []
    for g in range(_N_GROUP):
        seg = sfc[:, g * _GSZ:(g + 1) * _GSZ]
        il = jax.lax.broadcasted_iota(jnp.int32, (_T, _GSZ), 1)
        m1 = jnp.max(seg, axis=1, keepdims=True)
        fi = jnp.min(jnp.where(seg == m1, il, 127), axis=1, keepdims=True)
        m2 = jnp.max(jnp.where(il == fi, _NEG, seg), axis=1, keepdims=True)
        gs.append(m1 + m2)
    group_scores = jnp.concatenate(gs, axis=1)

    gil = jax.lax.broadcasted_iota(jnp.int32, (_T, _N_GROUP), 1)
    gmask = jnp.zeros((_T, _N_GROUP), jnp.float32)
    gtmp = group_scores
    for _ in range(_TOPK_GROUP):
        m = jnp.max(gtmp, axis=1, keepdims=True)
        fi = jnp.min(jnp.where(gtmp == m, gil, 127), axis=1, keepdims=True)
        sel = gil == fi
        gmask = jnp.where(sel, 1.0, gmask)
        gtmp = jnp.where(sel, _NEG, gtmp)

    smask = jnp.concatenate(
        [jnp.broadcast_to(gmask[:, g:g + 1], (_T, _GSZ)) for g in range(_N_GROUP)],
        axis=1)
    masked = jnp.where(smask > 0, sfc, 0.0)

    comb = jnp.zeros((_T, _E), jnp.float32)
    wsum = jnp.zeros((_T, 1), jnp.float32)
    for _ in range(_TOP_K):
        m = jnp.max(masked, axis=1, keepdims=True)
        fi = jnp.min(jnp.where(masked == m, lane, 9999), axis=1, keepdims=True)
        sel = lane == fi
        w = jnp.sum(jnp.where(sel, scores, 0.0), axis=1, keepdims=True)
        comb = comb + jnp.where(sel, w, 0.0)
        wsum = wsum + w
        masked = jnp.where(sel, _NEG, masked)
    return comb * (_SCALING / (wsum + 1e-20))


_EPG = 4


def _moe_body(x_ref, rw_ref, eb_ref, up_ref, dn_ref,
              su_ref, sd_ref, out_ref, comb_ref):
    e = pl.program_id(0)
    x = x_ref[...]

    @pl.when(e == 0)
    def _init():
        comb_ref[...] = _routing(x, rw_ref[...], eb_ref[...])
        out_ref[...] = jnp.zeros_like(out_ref)

    xb = x.astype(jnp.bfloat16)

    @pl.when((e >= _SH_FIRST) & (e < _SH_FIRST + _SH_STEPS))
    def _shared():
        hs = jnp.maximum(jax.lax.dot_general(
            xb, su_ref[...].astype(jnp.bfloat16), (((1,), (1,)), ((), ())),
            preferred_element_type=jnp.float32), 0.0)
        out_ref[...] += jax.lax.dot_general(
            hs.astype(jnp.bfloat16), sd_ref[...].astype(jnp.bfloat16),
            (((1,), (1,)), ((), ())),
            preferred_element_type=jnp.float32)

    lane = jax.lax.broadcasted_iota(jnp.int32, (_T, _E), 1)
    acc = out_ref[...]
    for j in range(_EPG):
        ej = e * _EPG + j
        c = jnp.sum(jnp.where(lane == ej, comb_ref[...], 0.0),
                    axis=1, keepdims=True)
        h = jnp.maximum(jax.lax.dot_general(
            xb, up_ref[j].astype(jnp.bfloat16), (((1,), (1,)), ((), ())),
            preferred_element_type=jnp.float32), 0.0)
        acc += jax.lax.dot_general(
            (h * c).astype(jnp.bfloat16), dn_ref[j].astype(jnp.bfloat16),
            (((1,), (1,)), ((), ())),
            preferred_element_type=jnp.float32)
    out_ref[...] = acc


def kernel(hidden_states, router_weight, up_w, down_w,
           shared_up_w, shared_down_w, e_bias):
    x = hidden_states.reshape(_T, _H)
    eb = e_bias.reshape(1, _E)

    out = pl.pallas_call(
        _moe_body,
        grid=(_E // _EPG,),
        in_specs=[
            pl.BlockSpec((_T, _H), lambda e: (0, 0)),
            pl.BlockSpec((_E, _H), lambda e: (0, 0)),
            pl.BlockSpec((1, _E), lambda e: (0, 0)),
            pl.BlockSpec((_EPG, _I_MOE, _H), lambda e: (e, 0, 0)),
            pl.BlockSpec((_EPG, _H, _I_MOE), lambda e: (e, 0, 0)),
            pl.BlockSpec((_SH_CHUNK, _H),
                         lambda e: (jnp.clip(e - _SH_FIRST, 0,
                                             _SH_STEPS - 1), 0)),
            pl.BlockSpec((_H, _SH_CHUNK),
                         lambda e: (0, jnp.clip(e - _SH_FIRST, 0,
                                                _SH_STEPS - 1))),
        ],
        out_specs=pl.BlockSpec((_T, _H), lambda e: (0, 0)),
        out_shape=jax.ShapeDtypeStruct((_T, _H), jnp.float32),
        scratch_shapes=[pltpu.VMEM((_T, _E), jnp.float32)],
    )(x, router_weight, eb, up_w, down_w, shared_up_w, shared_down_w)

    return out.reshape(_B, _S, _H)
